# Initial kernel scaffold; baseline (speedup 1.0000x reference)
import jax
import jax.numpy as jnp
from jax.experimental import pallas as pl


def kernel(points_xyz, features, W1, b1, W2, b2, Ws, bs, Wi1, bi1, Wi2, bi2):
    raise NotImplementedError("write your pallas kernel here")


# speedup vs baseline: 11.4319x; 11.4319x over previous
import functools

import jax
import jax.numpy as jnp
import numpy as np
from jax import lax
from jax.experimental import pallas as pl
from jax.experimental.pallas import tpu as pltpu
from jax.experimental.pallas import tpu_sc as plsc

B, N, C = 2, 8192, 64
NPOINT, K = 1024, 32
R2 = np.float32(0.1 * 0.1)
NR, NC_ = 64, 128
D = 128
NWORK = 32
CPW = (B * NPOINT) // NWORK


def _fps_body(xyz_ref, idx_ref, nxyz_ref):
    x = xyz_ref[:, 0]
    y = xyz_ref[:, 1]
    z = xyz_ref[:, 2]
    lin = (lax.broadcasted_iota(jnp.int32, (B, NR, NC_), 1) * NC_
           + lax.broadcasted_iota(jnp.int32, (B, NR, NC_), 2))
    idx_ref[pl.ds(0, 1), :] = jnp.zeros((1, B), jnp.int32)

    def extract(last):
        m = lin == last
        px = jnp.sum(jnp.where(m, x, 0.0), axis=(1, 2), keepdims=True)
        py = jnp.sum(jnp.where(m, y, 0.0), axis=(1, 2), keepdims=True)
        pz = jnp.sum(jnp.where(m, z, 0.0), axis=(1, 2), keepdims=True)
        return px, py, pz

    def store_xyz(i, px, py, pz):
        for b in range(B):
            for j, comp in enumerate((px, py, pz)):
                nxyz_ref[pl.ds(i, 1), pl.ds(b * 3 + j, 1)] = comp[b]

    def body(i, carry):
        dists, last = carry
        px, py, pz = extract(last)
        store_xyz(i - 1, px, py, pz)
        dx = x - px
        dy = y - py
        dz = z - pz
        d = dx * dx + dy * dy + dz * dz
        dists = jnp.minimum(dists, d)
        mx = jnp.max(dists, axis=(1, 2), keepdims=True)
        cand = jnp.where(dists == mx, lin, N)
        nidx = jnp.min(cand, axis=(1, 2), keepdims=True)
        idx_ref[pl.ds(i, 1), :] = nidx.reshape(1, B)
        return dists, nidx

    dists0 = jnp.full((B, NR, NC_), 1e10, jnp.float32)
    last0 = jnp.zeros((B, 1, 1), jnp.int32)
    _, last = lax.fori_loop(1, NPOINT, body, (dists0, last0))
    px, py, pz = extract(last)
    store_xyz(NPOINT - 1, px, py, pz)


def _fps(xyz_r):
    return pl.pallas_call(
        _fps_body,
        out_shape=[jax.ShapeDtypeStruct((NPOINT, B), jnp.int32),
                   jax.ShapeDtypeStruct((NPOINT, 3 * B), jnp.float32)],
    )(xyz_r)


def _sc_body(xf_hbm, yf_hbm, zf_hbm, cxf_hbm, cyf_hbm, czf_hbm,
             table_hbm, out_hbm, cnt_hbm,
             xv, yv, zv, cxb, cyb, czb, idxa, idxb, cntbuf, rows, sem):
    wid = lax.axis_index("s") * 2 + lax.axis_index("c")
    b = wid // 16
    base = wid * CPW

    pltpu.sync_copy(xf_hbm.at[pl.ds(b * N, N)], xv)
    pltpu.sync_copy(yf_hbm.at[pl.ds(b * N, N)], yv)
    pltpu.sync_copy(zf_hbm.at[pl.ds(b * N, N)], zv)

    pltpu.sync_copy(cxf_hbm.at[pl.ds(base, CPW)], cxb)
    pltpu.sync_copy(cyf_hbm.at[pl.ds(base, CPW)], cyb)
    pltpu.sync_copy(czf_hbm.at[pl.ds(base, CPW)], czb)

    lanes = lax.broadcasted_iota(jnp.int32, (16,), 0)
    zeros16 = jnp.zeros((16,), jnp.int32)
    gbase = b * N

    def per_centroid(c, _):
        cvec = jnp.full((16,), c, jnp.int32)
        cx = plsc.load_gather(cxb, [cvec])
        cy = plsc.load_gather(cyb, [cvec])
        cz = plsc.load_gather(czb, [cvec])
        idxa[...] = zeros16
        idxb[...] = zeros16

        def scan_chunk(s, cursor):
            off = s * 16
            px = xv[pl.ds(off, 16)]
            py = yv[pl.ds(off, 16)]
            pz = zv[pl.ds(off, 16)]
            dx = px - cx
            dy = py - cy
            dz = pz - cz
            d2 = dx * dx + dy * dy + dz * dz
            within = d2 < R2
            slot = cursor + plsc.cumsum(within.astype(jnp.int32)) - 1
            gidx = lanes + (off + gbase)
            keep_a = jnp.logical_and(within, slot < 16)
            plsc.store_scatter(idxa, [slot], gidx, mask=keep_a)
            keep_b = jnp.logical_and(within,
                                     jnp.logical_and(slot >= 16, slot < K))
            plsc.store_scatter(idxb, [slot - 16], gidx, mask=keep_b)
            cnt = plsc.all_reduce_population_count(within)
            return cursor + cnt

        cursor = lax.fori_loop(0, N // 16, scan_chunk,
                               jnp.zeros((16,), jnp.int32))

        plsc.store_scatter(cntbuf, [cvec], jnp.minimum(cursor, K),
                           mask=lanes == 0)

        da = pltpu.async_copy(table_hbm.at[idxa], rows.at[pl.ds(0, 16)], sem)
        db = pltpu.async_copy(table_hbm.at[idxb], rows.at[pl.ds(16, 16)], sem)
        da.wait()
        db.wait()
        pltpu.sync_copy(rows, out_hbm.at[pl.ds((base + c) * K, K)])
        return 0

    lax.fori_loop(0, CPW, per_centroid, 0)
    pltpu.sync_copy(cntbuf, cnt_hbm.at[pl.ds(base, CPW)])


def _sc_group(xf, yf, zf, cxf, cyf, czf, table):
    mesh = plsc.VectorSubcoreMesh(core_axis_name="c", subcore_axis_name="s",
                                  num_cores=2, num_subcores=16)
    kern = pl.kernel(
        _sc_body,
        out_type=[jax.ShapeDtypeStruct((B * NPOINT * K, D), jnp.float32),
                  jax.ShapeDtypeStruct((B * NPOINT,), jnp.int32)],
        mesh=mesh,
        compiler_params=pltpu.CompilerParams(needs_layout_passes=False),
        scratch_types=[
            pltpu.VMEM((N,), jnp.float32),
            pltpu.VMEM((N,), jnp.float32),
            pltpu.VMEM((N,), jnp.float32),
            pltpu.VMEM((CPW,), jnp.float32),
            pltpu.VMEM((CPW,), jnp.float32),
            pltpu.VMEM((CPW,), jnp.float32),
            pltpu.VMEM((16,), jnp.int32),
            pltpu.VMEM((16,), jnp.int32),
            pltpu.VMEM((CPW,), jnp.int32),
            pltpu.VMEM((K, D), jnp.float32),
            pltpu.SemaphoreType.DMA,
        ],
    )
    return kern(xf, yf, zf, cxf, cyf, czf, table)


CB = 16
RB = CB * K
NBLK = (B * NPOINT) // CB


def _mlp_body(rows_ref, cent_ref, cnt_ref, w1_ref, b1_ref, w2_ref, b2_ref,
              ws_ref, bs_ref, wi1_ref, bi1_ref, wi2_ref, bi2_ref, out_ref):
    g = rows_ref[...].reshape(CB, K, D) - cent_ref[...][:, None, :]
    g = g.reshape(RB, D)
    dot = functools.partial(jnp.dot, preferred_element_type=jnp.float32)
    h1 = jnp.maximum(dot(g, w1_ref[...]) + b1_ref[...], 0.0)
    h2 = dot(h1, w2_ref[...]) + b2_ref[...]
    hs = dot(g, ws_ref[...]) + bs_ref[...]
    h = jnp.maximum(h2 + hs, 0.0)
    hi1 = jnp.maximum(dot(h, wi1_ref[...]) + bi1_ref[...], 0.0)
    hi2 = dot(hi1, wi2_ref[...]) + bi2_ref[...]
    h = jnp.maximum(hi2 + h, 0.0)
    h3 = h.reshape(CB, K, 128)
    kio = lax.broadcasted_iota(jnp.int32, (CB, K, 128), 1)
    valid = kio < cnt_ref[...][:, :, None]
    out_ref[...] = jnp.max(jnp.where(valid, h3, -jnp.inf), axis=1)


def _mlp(rows, cent_pad, cnt2d, weights):
    full = lambda shape: pl.BlockSpec(shape, lambda i: (0, 0))
    return pl.pallas_call(
        _mlp_body,
        grid=(NBLK,),
        in_specs=[
            pl.BlockSpec((RB, D), lambda i: (i, 0)),
            pl.BlockSpec((CB, D), lambda i: (i, 0)),
            pl.BlockSpec((CB, 1), lambda i: (i, 0)),
            full((D, 128)), full((1, 128)),
            full((128, 128)), full((1, 128)),
            full((D, 128)), full((1, 128)),
            full((128, 256)), full((1, 256)),
            full((256, 128)), full((1, 128)),
        ],
        out_specs=pl.BlockSpec((CB, 128), lambda i: (i, 0)),
        out_shape=jax.ShapeDtypeStruct((B * NPOINT, 128), jnp.float32),
    )(rows, cent_pad, cnt2d, *weights)


def kernel(points_xyz, features, W1, b1, W2, b2, Ws, bs, Wi1, bi1, Wi2, bi2):
    xyzt = points_xyz.transpose(0, 2, 1)
    xyz_r = xyzt.reshape(B, 3, NR, NC_)

    idx_t, nxyz_t = _fps(xyz_r)
    indices = idx_t.T
    new_xyz = nxyz_t.reshape(NPOINT, B, 3).transpose(1, 0, 2)

    table = jnp.concatenate(
        [points_xyz, features.transpose(0, 2, 1),
         jnp.zeros((B, N, D - 3 - C), jnp.float32)], axis=-1
    ).reshape(B * N, D)

    xf = xyzt[:, 0].reshape(B * N)
    yf = xyzt[:, 1].reshape(B * N)
    zf = xyzt[:, 2].reshape(B * N)
    cxf = new_xyz[:, :, 0].reshape(B * NPOINT)
    cyf = new_xyz[:, :, 1].reshape(B * NPOINT)
    czf = new_xyz[:, :, 2].reshape(B * NPOINT)
    rows, counts = _sc_group(xf, yf, zf, cxf, cyf, czf, table)
    nxyz_flat = new_xyz.reshape(B * NPOINT, 3)

    cent_pad = jnp.concatenate(
        [nxyz_flat, jnp.zeros((B * NPOINT, D - 3), jnp.float32)], axis=-1)

    w1p = jnp.pad(W1, ((0, 0), (0, D - 3 - C))).T
    wsp = jnp.pad(Ws, ((0, 0), (0, D - 3 - C))).T
    weights = (w1p, b1.reshape(1, 128), W2.T, b2.reshape(1, 128),
               wsp, bs.reshape(1, 128), Wi1.T, bi1.reshape(1, 256),
               Wi2.T, bi2.reshape(1, 128))

    feats = _mlp(rows, cent_pad, counts.reshape(B * NPOINT, 1),
                 weights)
    new_features = feats.reshape(B, NPOINT, 128).transpose(0, 2, 1)
    return new_xyz, new_features, indices


# speedup vs baseline: 14.0533x; 1.2293x over previous
import functools

import jax
import jax.numpy as jnp
import numpy as np
from jax import lax
from jax.experimental import pallas as pl
from jax.experimental.pallas import tpu as pltpu
from jax.experimental.pallas import tpu_sc as plsc

B, N, C = 2, 8192, 64
NPOINT, K = 1024, 32
R2 = np.float32(0.1 * 0.1)
NR, NC_ = 64, 128
D = 128
NWORK = 32
CPW = (B * NPOINT) // NWORK


def _fps_body(xyz_ref, idx_ref, nxyz_ref):
    x = xyz_ref[:, 0]
    y = xyz_ref[:, 1]
    z = xyz_ref[:, 2]
    lin = (lax.broadcasted_iota(jnp.int32, (B, NR, NC_), 1) * NC_
           + lax.broadcasted_iota(jnp.int32, (B, NR, NC_), 2))
    idx_ref[pl.ds(0, 1), :] = jnp.zeros((1, B), jnp.int32)

    def extract(last):
        m = lin == last
        px = jnp.sum(jnp.where(m, x, 0.0), axis=(1, 2), keepdims=True)
        py = jnp.sum(jnp.where(m, y, 0.0), axis=(1, 2), keepdims=True)
        pz = jnp.sum(jnp.where(m, z, 0.0), axis=(1, 2), keepdims=True)
        return px, py, pz

    def store_xyz(i, px, py, pz):
        for b in range(B):
            for j, comp in enumerate((px, py, pz)):
                nxyz_ref[pl.ds(i, 1), pl.ds(b * 3 + j, 1)] = comp[b]

    def body(i, carry):
        dists, last = carry
        px, py, pz = extract(last)
        store_xyz(i - 1, px, py, pz)
        dx = x - px
        dy = y - py
        dz = z - pz
        d = dx * dx + dy * dy + dz * dz
        dists = jnp.minimum(dists, d)
        mx = jnp.max(dists, axis=(1, 2), keepdims=True)
        cand = jnp.where(dists == mx, lin, N)
        nidx = jnp.min(cand, axis=(1, 2), keepdims=True)
        idx_ref[pl.ds(i, 1), :] = nidx.reshape(1, B)
        return dists, nidx

    dists0 = jnp.full((B, NR, NC_), 1e10, jnp.float32)
    last0 = jnp.zeros((B, 1, 1), jnp.int32)
    _, last = lax.fori_loop(1, NPOINT, body, (dists0, last0))
    px, py, pz = extract(last)
    store_xyz(NPOINT - 1, px, py, pz)


def _fps(xyz_r):
    return pl.pallas_call(
        _fps_body,
        out_shape=[jax.ShapeDtypeStruct((NPOINT, B), jnp.int32),
                   jax.ShapeDtypeStruct((NPOINT, 3 * B), jnp.float32)],
    )(xyz_r)


def _sc_body(xf_hbm, yf_hbm, zf_hbm, cxf_hbm, cyf_hbm, czf_hbm,
             table_hbm, out_hbm, cnt_hbm,
             xv, yv, zv, cxb, cyb, czb, idxa, idxb, cntbuf, rows, sem):
    wid = lax.axis_index("s") * 2 + lax.axis_index("c")
    b = wid // 16
    base = wid * CPW

    pltpu.sync_copy(xf_hbm.at[pl.ds(b * N, N)], xv)
    pltpu.sync_copy(yf_hbm.at[pl.ds(b * N, N)], yv)
    pltpu.sync_copy(zf_hbm.at[pl.ds(b * N, N)], zv)

    pltpu.sync_copy(cxf_hbm.at[pl.ds(base, CPW)], cxb)
    pltpu.sync_copy(cyf_hbm.at[pl.ds(base, CPW)], cyb)
    pltpu.sync_copy(czf_hbm.at[pl.ds(base, CPW)], czb)

    lanes = lax.broadcasted_iota(jnp.int32, (16,), 0)
    zeros16 = jnp.zeros((16,), jnp.int32)
    gbase = b * N

    def per_centroid(c, _):
        cvec = jnp.full((16,), c, jnp.int32)
        cx = plsc.load_gather(cxb, [cvec])
        cy = plsc.load_gather(cyb, [cvec])
        cz = plsc.load_gather(czb, [cvec])
        idxa[...] = zeros16
        idxb[...] = zeros16

        UNROLL = 4

        def scan_super(t, cursor):
            off = t * (16 * UNROLL)
            masks = []
            for u in range(UNROLL):
                px = xv[pl.ds(off + u * 16, 16)]
                py = yv[pl.ds(off + u * 16, 16)]
                pz = zv[pl.ds(off + u * 16, 16)]
                dx = px - cx
                dy = py - cy
                dz = pz - cz
                d2 = dx * dx + dy * dy + dz * dz
                masks.append(d2 < R2)
            comb = masks[0]
            for u in range(1, UNROLL):
                comb = jnp.logical_or(comb, masks[u])
            any_w = jnp.any(comb)

            def compact(cur):
                for u in range(UNROLL):
                    w = masks[u]
                    slot = cur + plsc.cumsum(w.astype(jnp.int32)) - 1
                    gidx = lanes + (off + u * 16 + gbase)
                    keep_a = jnp.logical_and(w, slot < 16)
                    plsc.store_scatter(idxa, [slot], gidx, mask=keep_a)
                    keep_b = jnp.logical_and(
                        w, jnp.logical_and(slot >= 16, slot < K))
                    plsc.store_scatter(idxb, [slot - 16], gidx, mask=keep_b)
                    cur = cur + plsc.all_reduce_population_count(w)
                return cur

            return lax.cond(any_w, compact, lambda cur: cur, cursor)

        cursor = lax.fori_loop(0, N // (16 * UNROLL), scan_super,
                               jnp.zeros((16,), jnp.int32))

        plsc.store_scatter(cntbuf, [cvec], jnp.minimum(cursor, K),
                           mask=lanes == 0)

        da = pltpu.async_copy(table_hbm.at[idxa], rows.at[pl.ds(0, 16)], sem)
        db = pltpu.async_copy(table_hbm.at[idxb], rows.at[pl.ds(16, 16)], sem)
        da.wait()
        db.wait()
        pltpu.sync_copy(rows, out_hbm.at[pl.ds((base + c) * K, K)])
        return 0

    lax.fori_loop(0, CPW, per_centroid, 0)
    pltpu.sync_copy(cntbuf, cnt_hbm.at[pl.ds(base, CPW)])


def _sc_group(xf, yf, zf, cxf, cyf, czf, table):
    mesh = plsc.VectorSubcoreMesh(core_axis_name="c", subcore_axis_name="s",
                                  num_cores=2, num_subcores=16)
    kern = pl.kernel(
        _sc_body,
        out_type=[jax.ShapeDtypeStruct((B * NPOINT * K, D), jnp.float32),
                  jax.ShapeDtypeStruct((B * NPOINT,), jnp.int32)],
        mesh=mesh,
        compiler_params=pltpu.CompilerParams(needs_layout_passes=False),
        scratch_types=[
            pltpu.VMEM((N,), jnp.float32),
            pltpu.VMEM((N,), jnp.float32),
            pltpu.VMEM((N,), jnp.float32),
            pltpu.VMEM((CPW,), jnp.float32),
            pltpu.VMEM((CPW,), jnp.float32),
            pltpu.VMEM((CPW,), jnp.float32),
            pltpu.VMEM((16,), jnp.int32),
            pltpu.VMEM((16,), jnp.int32),
            pltpu.VMEM((CPW,), jnp.int32),
            pltpu.VMEM((K, D), jnp.float32),
            pltpu.SemaphoreType.DMA,
        ],
    )
    return kern(xf, yf, zf, cxf, cyf, czf, table)


CB = 16
RB = CB * K
NBLK = (B * NPOINT) // CB


def _mlp_body(rows_ref, cent_ref, cnt_ref, w1_ref, b1_ref, w2_ref, b2_ref,
              ws_ref, bs_ref, wi1_ref, bi1_ref, wi2_ref, bi2_ref, out_ref):
    g = rows_ref[...].reshape(CB, K, D) - cent_ref[...][:, None, :]
    g = g.reshape(RB, D)
    dot = functools.partial(jnp.dot, preferred_element_type=jnp.float32)
    h1 = jnp.maximum(dot(g, w1_ref[...]) + b1_ref[...], 0.0)
    h2 = dot(h1, w2_ref[...]) + b2_ref[...]
    hs = dot(g, ws_ref[...]) + bs_ref[...]
    h = jnp.maximum(h2 + hs, 0.0)
    hi1 = jnp.maximum(dot(h, wi1_ref[...]) + bi1_ref[...], 0.0)
    hi2 = dot(hi1, wi2_ref[...]) + bi2_ref[...]
    h = jnp.maximum(hi2 + h, 0.0)
    h3 = h.reshape(CB, K, 128)
    kio = lax.broadcasted_iota(jnp.int32, (CB, K, 128), 1)
    valid = kio < cnt_ref[...][:, :, None]
    out_ref[...] = jnp.max(jnp.where(valid, h3, -jnp.inf), axis=1)


def _mlp(rows, cent_pad, cnt2d, weights):
    full = lambda shape: pl.BlockSpec(shape, lambda i: (0, 0))
    return pl.pallas_call(
        _mlp_body,
        grid=(NBLK,),
        in_specs=[
            pl.BlockSpec((RB, D), lambda i: (i, 0)),
            pl.BlockSpec((CB, D), lambda i: (i, 0)),
            pl.BlockSpec((CB, 1), lambda i: (i, 0)),
            full((D, 128)), full((1, 128)),
            full((128, 128)), full((1, 128)),
            full((D, 128)), full((1, 128)),
            full((128, 256)), full((1, 256)),
            full((256, 128)), full((1, 128)),
        ],
        out_specs=pl.BlockSpec((CB, 128), lambda i: (i, 0)),
        out_shape=jax.ShapeDtypeStruct((B * NPOINT, 128), jnp.float32),
    )(rows, cent_pad, cnt2d, *weights)


def kernel(points_xyz, features, W1, b1, W2, b2, Ws, bs, Wi1, bi1, Wi2, bi2):
    xyzt = points_xyz.transpose(0, 2, 1)
    xyz_r = xyzt.reshape(B, 3, NR, NC_)

    idx_t, nxyz_t = _fps(xyz_r)
    indices = idx_t.T
    new_xyz = nxyz_t.reshape(NPOINT, B, 3).transpose(1, 0, 2)

    table = jnp.concatenate(
        [points_xyz, features.transpose(0, 2, 1),
         jnp.zeros((B, N, D - 3 - C), jnp.float32)], axis=-1
    ).reshape(B * N, D)

    xf = xyzt[:, 0].reshape(B * N)
    yf = xyzt[:, 1].reshape(B * N)
    zf = xyzt[:, 2].reshape(B * N)
    cxf = new_xyz[:, :, 0].reshape(B * NPOINT)
    cyf = new_xyz[:, :, 1].reshape(B * NPOINT)
    czf = new_xyz[:, :, 2].reshape(B * NPOINT)
    rows, counts = _sc_group(xf, yf, zf, cxf, cyf, czf, table)
    nxyz_flat = new_xyz.reshape(B * NPOINT, 3)

    cent_pad = jnp.concatenate(
        [nxyz_flat, jnp.zeros((B * NPOINT, D - 3), jnp.float32)], axis=-1)

    w1p = jnp.pad(W1, ((0, 0), (0, D - 3 - C))).T
    wsp = jnp.pad(Ws, ((0, 0), (0, D - 3 - C))).T
    weights = (w1p, b1.reshape(1, 128), W2.T, b2.reshape(1, 128),
               wsp, bs.reshape(1, 128), Wi1.T, bi1.reshape(1, 256),
               Wi2.T, bi2.reshape(1, 128))

    feats = _mlp(rows, cent_pad, counts.reshape(B * NPOINT, 1),
                 weights)
    new_features = feats.reshape(B, NPOINT, 128).transpose(0, 2, 1)
    return new_xyz, new_features, indices
